# baseline (speedup 1.0000x reference)
import jax
import jax.numpy as jnp
from jax.experimental import pallas as pl
from jax.experimental.pallas import tpu as pltpu

_B, _C, _H, _W = 4, 96, 256, 256
_BC = 8
_GJ = _C // _BC
_N = _B * _GJ
_K = 24
_SCALE = 127.0


def _blk(n):
    return (n // _GJ, n % _GJ, 0, 0)


def _outer(x_hbm, q_hbm, min_ref, max_ref, cache, acc_min, acc_max):
    def body0(idxs, x_vmem):
        (n,) = idxs

        @pl.when(n == 0)
        def _init():
            acc_min[...] = jnp.full_like(acc_min, jnp.inf)
            acc_max[...] = jnp.full_like(acc_max, -jnp.inf)

        x = x_vmem[...]
        xv = x.reshape(_BC * _H, _W)
        acc_min[...] = jnp.minimum(acc_min[...], jnp.min(xv, axis=0, keepdims=True))
        acc_max[...] = jnp.maximum(acc_max[...], jnp.max(xv, axis=0, keepdims=True))

        @pl.when(n < _K)
        def _stash():
            cache[pl.ds(n, 1)] = x

    pltpu.emit_pipeline(
        body0,
        grid=(_N,),
        in_specs=[pl.BlockSpec((1, _BC, _H, _W), _blk)],
        _explicit_indices=True,
    )(x_hbm)

    x_min = jnp.min(acc_min[...])
    x_max = jnp.max(acc_max[...])
    min_ref[0, 0] = x_min
    max_ref[0, 0] = x_max

    def quant(x):
        xn = (x - x_min) / (x_max - x_min) * 2.0 - 1.0
        return jnp.round(xn * _SCALE) / _SCALE

    def body1a(idxs, o_vmem):
        (n,) = idxs
        o_vmem[...] = quant(cache[pl.ds(n, 1)])

    pltpu.emit_pipeline(
        body1a,
        grid=(_K,),
        out_specs=[pl.BlockSpec((1, _BC, _H, _W), _blk)],
        _explicit_indices=True,
    )(q_hbm)

    def body1b(x_vmem, o_vmem):
        o_vmem[...] = quant(x_vmem[...])

    pltpu.emit_pipeline(
        body1b,
        grid=(_N - _K,),
        in_specs=[pl.BlockSpec((1, _BC, _H, _W), lambda n: _blk(n + _K))],
        out_specs=[pl.BlockSpec((1, _BC, _H, _W), lambda n: _blk(n + _K))],
    )(x_hbm, q_hbm)


def kernel(x_ll):
    q, x_min, x_max = pl.pallas_call(
        _outer,
        in_specs=[pl.BlockSpec(memory_space=pl.ANY)],
        out_specs=[
            pl.BlockSpec(memory_space=pl.ANY),
            pl.BlockSpec(memory_space=pltpu.SMEM),
            pl.BlockSpec(memory_space=pltpu.SMEM),
        ],
        out_shape=[
            jax.ShapeDtypeStruct((_B, _C, _H, _W), jnp.float32),
            jax.ShapeDtypeStruct((1, 1), jnp.float32),
            jax.ShapeDtypeStruct((1, 1), jnp.float32),
        ],
        scratch_shapes=[
            pltpu.VMEM((_K, _BC, _H, _W), jnp.float32),
            pltpu.VMEM((1, _W), jnp.float32),
            pltpu.VMEM((1, _W), jnp.float32),
        ],
    )(x_ll)

    return (q, x_min.reshape(()), x_max.reshape(()))


# speedup vs baseline: 1.0370x; 1.0370x over previous
import jax
import jax.numpy as jnp
from jax.experimental import pallas as pl
from jax.experimental.pallas import tpu as pltpu

_B, _C, _H, _W = 4, 96, 256, 256
_BC = 8
_GJ = _C // _BC
_N = _B * _GJ
_K = 24
_SCALE = 127.0


def _blk(n):
    return (n // _GJ, n % _GJ, 0, 0)


def _outer(x_hbm, q_hbm, min_ref, max_ref, cache, acc_min, acc_max):
    def body0(idxs, x_vmem):
        (n,) = idxs

        @pl.when(n == 0)
        def _init():
            acc_min[...] = jnp.full_like(acc_min, jnp.inf)
            acc_max[...] = jnp.full_like(acc_max, -jnp.inf)

        x = x_vmem[...]
        xv = x.reshape(_BC * _H, _W)
        acc_min[...] = jnp.minimum(acc_min[...], jnp.min(xv, axis=0, keepdims=True))
        acc_max[...] = jnp.maximum(acc_max[...], jnp.max(xv, axis=0, keepdims=True))

        @pl.when(n < _K)
        def _stash():
            cache[pl.ds(n, 1)] = x

    pltpu.emit_pipeline(
        body0,
        grid=(_N,),
        in_specs=[pl.BlockSpec((1, _BC, _H, _W), _blk)],
        _explicit_indices=True,
    )(x_hbm)

    x_min = jnp.min(acc_min[...])
    x_max = jnp.max(acc_max[...])
    min_ref[0, 0] = x_min
    max_ref[0, 0] = x_max

    a = (2.0 * _SCALE) / (x_max - x_min)
    b = x_min * a + _SCALE
    inv = 1.0 / _SCALE

    def quant(x):
        return jnp.round(x * a - b) * inv

    def body1a(idxs, o_vmem):
        (n,) = idxs
        o_vmem[...] = quant(cache[pl.ds(n, 1)])

    pltpu.emit_pipeline(
        body1a,
        grid=(_K,),
        out_specs=[pl.BlockSpec((1, _BC, _H, _W), _blk)],
        _explicit_indices=True,
    )(q_hbm)

    def body1b(x_vmem, o_vmem):
        o_vmem[...] = quant(x_vmem[...])

    pltpu.emit_pipeline(
        body1b,
        grid=(_N - _K,),
        in_specs=[pl.BlockSpec((1, _BC, _H, _W), lambda n: _blk(n + _K))],
        out_specs=[pl.BlockSpec((1, _BC, _H, _W), lambda n: _blk(n + _K))],
    )(x_hbm, q_hbm)


def kernel(x_ll):
    q, x_min, x_max = pl.pallas_call(
        _outer,
        in_specs=[pl.BlockSpec(memory_space=pl.ANY)],
        out_specs=[
            pl.BlockSpec(memory_space=pl.ANY),
            pl.BlockSpec(memory_space=pltpu.SMEM),
            pl.BlockSpec(memory_space=pltpu.SMEM),
        ],
        out_shape=[
            jax.ShapeDtypeStruct((_B, _C, _H, _W), jnp.float32),
            jax.ShapeDtypeStruct((1, 1), jnp.float32),
            jax.ShapeDtypeStruct((1, 1), jnp.float32),
        ],
        scratch_shapes=[
            pltpu.VMEM((_K, _BC, _H, _W), jnp.float32),
            pltpu.VMEM((1, _W), jnp.float32),
            pltpu.VMEM((1, _W), jnp.float32),
        ],
    )(x_ll)

    return (q, x_min.reshape(()), x_max.reshape(()))


# speedup vs baseline: 1.2347x; 1.1906x over previous
import jax
import jax.numpy as jnp
from jax.experimental import pallas as pl
from jax.experimental.pallas import tpu as pltpu

_B, _C, _H, _W = 4, 96, 256, 256
_BC = 16
_GJ = _C // _BC
_N = _B * _GJ
_K = 10
_SCALE = 127.0


def _blk(n):
    return (n // _GJ, n % _GJ, 0, 0)


def _outer(x_hbm, q_hbm, min_ref, max_ref, cache, acc_min, acc_max):
    def body0(idxs, x_vmem):
        (n,) = idxs

        @pl.when(n == 0)
        def _init():
            acc_min[...] = jnp.full_like(acc_min, jnp.inf)
            acc_max[...] = jnp.full_like(acc_max, -jnp.inf)

        x = x_vmem[...]
        xv = x.reshape(_BC * _H, _W)
        acc_min[...] = jnp.minimum(acc_min[...], jnp.min(xv, axis=0, keepdims=True))
        acc_max[...] = jnp.maximum(acc_max[...], jnp.max(xv, axis=0, keepdims=True))

        @pl.when(n < _K)
        def _stash():
            cache[pl.ds(n, 1)] = x

    pltpu.emit_pipeline(
        body0,
        grid=(_N,),
        in_specs=[pl.BlockSpec((1, _BC, _H, _W), _blk)],
        _explicit_indices=True,
    )(x_hbm)

    x_min = jnp.min(acc_min[...])
    x_max = jnp.max(acc_max[...])
    min_ref[0, 0] = x_min
    max_ref[0, 0] = x_max

    a = (2.0 * _SCALE) / (x_max - x_min)
    b = x_min * a + _SCALE
    inv = 1.0 / _SCALE

    def quant(x):
        return jnp.round(x * a - b) * inv

    def body1a(idxs, o_vmem):
        (n,) = idxs
        o_vmem[...] = quant(cache[pl.ds(n, 1)])

    pltpu.emit_pipeline(
        body1a,
        grid=(_K,),
        out_specs=[pl.BlockSpec((1, _BC, _H, _W), _blk)],
        _explicit_indices=True,
    )(q_hbm)

    def body1b(x_vmem, o_vmem):
        o_vmem[...] = quant(x_vmem[...])

    pltpu.emit_pipeline(
        body1b,
        grid=(_N - _K,),
        in_specs=[pl.BlockSpec((1, _BC, _H, _W), lambda n: _blk(n + _K))],
        out_specs=[pl.BlockSpec((1, _BC, _H, _W), lambda n: _blk(n + _K))],
    )(x_hbm, q_hbm)


def kernel(x_ll):
    q, x_min, x_max = pl.pallas_call(
        _outer,
        in_specs=[pl.BlockSpec(memory_space=pl.ANY)],
        out_specs=[
            pl.BlockSpec(memory_space=pl.ANY),
            pl.BlockSpec(memory_space=pltpu.SMEM),
            pl.BlockSpec(memory_space=pltpu.SMEM),
        ],
        out_shape=[
            jax.ShapeDtypeStruct((_B, _C, _H, _W), jnp.float32),
            jax.ShapeDtypeStruct((1, 1), jnp.float32),
            jax.ShapeDtypeStruct((1, 1), jnp.float32),
        ],
        scratch_shapes=[
            pltpu.VMEM((_K, _BC, _H, _W), jnp.float32),
            pltpu.VMEM((1, _W), jnp.float32),
            pltpu.VMEM((1, _W), jnp.float32),
        ],
    )(x_ll)

    return (q, x_min.reshape(()), x_max.reshape(()))


# speedup vs baseline: 1.3216x; 1.0705x over previous
import jax
import jax.numpy as jnp
from jax.experimental import pallas as pl
from jax.experimental.pallas import tpu as pltpu

_B, _C, _H, _W = 4, 96, 256, 256
_BC = 16
_GJ = _C // _BC
_N = _B * _GJ
_K = 10
_SCALE = 127.0


def _blk(n):
    return (n // _GJ, n % _GJ, 0, 0)


def _outer(x_hbm, q_hbm, min_ref, max_ref, cache, acc_min, acc_max):
    def body0(idxs, x_vmem):
        (n,) = idxs

        @pl.when(n == 0)
        def _init():
            acc_min[...] = jnp.full_like(acc_min, jnp.inf)
            acc_max[...] = jnp.full_like(acc_max, -jnp.inf)

        x = x_vmem[...]
        xv = x.reshape(_BC * _H, _W)
        acc_min[...] = jnp.minimum(acc_min[...], jnp.min(xv, axis=0, keepdims=True))
        acc_max[...] = jnp.maximum(acc_max[...], jnp.max(xv, axis=0, keepdims=True))

        @pl.when(n < _K)
        def _stash():
            cache[pl.ds(n, 1)] = x

    pltpu.emit_pipeline(
        body0,
        grid=(_N,),
        in_specs=[pl.BlockSpec((1, _BC, _H, _W), _blk,
                               pipeline_mode=pl.Buffered(buffer_count=3))],
        _explicit_indices=True,
    )(x_hbm)

    x_min = jnp.min(acc_min[...])
    x_max = jnp.max(acc_max[...])
    min_ref[0, 0] = x_min
    max_ref[0, 0] = x_max

    a = (2.0 * _SCALE) / (x_max - x_min)
    b = x_min * a + _SCALE
    inv = 1.0 / _SCALE

    def quant(x):
        return jnp.round(x * a - b) * inv

    def body1a(idxs, o_vmem):
        (n,) = idxs
        o_vmem[...] = quant(cache[pl.ds(n, 1)])

    pltpu.emit_pipeline(
        body1a,
        grid=(_K,),
        out_specs=[pl.BlockSpec((1, _BC, _H, _W), _blk)],
        _explicit_indices=True,
    )(q_hbm)

    def body1b(x_vmem, o_vmem):
        o_vmem[...] = quant(x_vmem[...])

    pltpu.emit_pipeline(
        body1b,
        grid=(_N - _K,),
        in_specs=[pl.BlockSpec((1, _BC, _H, _W), lambda n: _blk(n + _K))],
        out_specs=[pl.BlockSpec((1, _BC, _H, _W), lambda n: _blk(n + _K))],
    )(x_hbm, q_hbm)


def kernel(x_ll):
    q, x_min, x_max = pl.pallas_call(
        _outer,
        in_specs=[pl.BlockSpec(memory_space=pl.ANY)],
        out_specs=[
            pl.BlockSpec(memory_space=pl.ANY),
            pl.BlockSpec(memory_space=pltpu.SMEM),
            pl.BlockSpec(memory_space=pltpu.SMEM),
        ],
        out_shape=[
            jax.ShapeDtypeStruct((_B, _C, _H, _W), jnp.float32),
            jax.ShapeDtypeStruct((1, 1), jnp.float32),
            jax.ShapeDtypeStruct((1, 1), jnp.float32),
        ],
        scratch_shapes=[
            pltpu.VMEM((_K, _BC, _H, _W), jnp.float32),
            pltpu.VMEM((1, _W), jnp.float32),
            pltpu.VMEM((1, _W), jnp.float32),
        ],
    )(x_ll)

    return (q, x_min.reshape(()), x_max.reshape(()))


# speedup vs baseline: 1.3319x; 1.0077x over previous
import jax
import jax.numpy as jnp
from jax.experimental import pallas as pl
from jax.experimental.pallas import tpu as pltpu

_B, _C, _H, _W = 4, 96, 256, 256
_BC = 16
_GJ = _C // _BC
_N = _B * _GJ
_K = 10
_SCALE = 127.0


def _blk(n):
    return (n // _GJ, n % _GJ, 0, 0)


def _outer(x_hbm, q_hbm, min_ref, max_ref, cache, acc_min, acc_max):
    def body0(idxs, x_vmem):
        (n,) = idxs

        @pl.when(n == 0)
        def _init():
            acc_min[...] = jnp.full_like(acc_min, jnp.inf)
            acc_max[...] = jnp.full_like(acc_max, -jnp.inf)

        x = x_vmem[...]
        xv = x.reshape(_BC * _H, _W)
        acc_min[...] = jnp.minimum(acc_min[...], jnp.min(xv, axis=0, keepdims=True))
        acc_max[...] = jnp.maximum(acc_max[...], jnp.max(xv, axis=0, keepdims=True))

        @pl.when(n < _K)
        def _stash():
            cache[pl.ds(n, 1)] = x

    pltpu.emit_pipeline(
        body0,
        grid=(_N,),
        in_specs=[pl.BlockSpec((1, _BC, _H, _W), _blk,
                               pipeline_mode=pl.Buffered(buffer_count=3))],
        _explicit_indices=True,
    )(x_hbm)

    x_min = jnp.min(acc_min[...])
    x_max = jnp.max(acc_max[...])
    min_ref[0, 0] = x_min
    max_ref[0, 0] = x_max

    a = (2.0 * _SCALE) / (x_max - x_min)
    b = x_min * a + _SCALE
    inv = 1.0 / _SCALE

    def quant(x):
        return jnp.round(x * a - b) * inv

    def body1a(idxs, o_vmem):
        (n,) = idxs
        o_vmem[...] = quant(cache[pl.ds(n, 1)])

    pltpu.emit_pipeline(
        body1a,
        grid=(_K,),
        out_specs=[pl.BlockSpec((1, _BC, _H, _W), _blk)],
        _explicit_indices=True,
    )(q_hbm)

    def body1b(x_vmem, o_vmem):
        o_vmem[...] = quant(x_vmem[...])

    _H2 = _BC // 2
    _GJ2 = _C // _H2

    def _blk2(n):
        m = n + 2 * _K
        return (m // _GJ2, m % _GJ2, 0, 0)

    pltpu.emit_pipeline(
        body1b,
        grid=(2 * (_N - _K),),
        in_specs=[pl.BlockSpec((1, _H2, _H, _W), _blk2,
                               pipeline_mode=pl.Buffered(buffer_count=3))],
        out_specs=[pl.BlockSpec((1, _H2, _H, _W), _blk2)],
    )(x_hbm, q_hbm)


def kernel(x_ll):
    q, x_min, x_max = pl.pallas_call(
        _outer,
        in_specs=[pl.BlockSpec(memory_space=pl.ANY)],
        out_specs=[
            pl.BlockSpec(memory_space=pl.ANY),
            pl.BlockSpec(memory_space=pltpu.SMEM),
            pl.BlockSpec(memory_space=pltpu.SMEM),
        ],
        out_shape=[
            jax.ShapeDtypeStruct((_B, _C, _H, _W), jnp.float32),
            jax.ShapeDtypeStruct((1, 1), jnp.float32),
            jax.ShapeDtypeStruct((1, 1), jnp.float32),
        ],
        scratch_shapes=[
            pltpu.VMEM((_K, _BC, _H, _W), jnp.float32),
            pltpu.VMEM((1, _W), jnp.float32),
            pltpu.VMEM((1, _W), jnp.float32),
        ],
    )(x_ll)

    return (q, x_min.reshape(()), x_max.reshape(()))


# speedup vs baseline: 1.3466x; 1.0110x over previous
import jax
import jax.numpy as jnp
from jax.experimental import pallas as pl
from jax.experimental.pallas import tpu as pltpu

_B, _C, _H, _W = 4, 96, 256, 256
_BC = 16
_GJ = _C // _BC
_N = _B * _GJ
_K = 10
_SCALE = 127.0


def _blk(n):
    return (n // _GJ, n % _GJ, 0, 0)


def _outer(x_hbm, q_hbm, min_ref, max_ref, cache, acc_min, acc_max):
    def body0(idxs, x_vmem):
        (n,) = idxs

        @pl.when(n == 0)
        def _init():
            acc_min[...] = jnp.full_like(acc_min, jnp.inf)
            acc_max[...] = jnp.full_like(acc_max, -jnp.inf)

        x = x_vmem[...]
        xv = x.reshape(_BC * _H, _W)
        acc_min[...] = jnp.minimum(acc_min[...], jnp.min(xv, axis=0, keepdims=True))
        acc_max[...] = jnp.maximum(acc_max[...], jnp.max(xv, axis=0, keepdims=True))

        @pl.when(n < _K)
        def _stash():
            cache[pl.ds(n, 1)] = x

    pltpu.emit_pipeline(
        body0,
        grid=(_N,),
        in_specs=[pl.BlockSpec((1, _BC, _H, _W), _blk,
                               pipeline_mode=pl.Buffered(buffer_count=3))],
        _explicit_indices=True,
    )(x_hbm)

    x_min = jnp.min(acc_min[...])
    x_max = jnp.max(acc_max[...])
    min_ref[0, 0] = x_min
    max_ref[0, 0] = x_max

    a = (2.0 * _SCALE) / (x_max - x_min)
    b = x_min * a + _SCALE
    inv = 1.0 / _SCALE

    def quant(x):
        return jnp.round(x * a - b) * inv

    def body1a(idxs, o_vmem):
        (n,) = idxs
        o_vmem[...] = quant(cache[pl.ds(n, 1)])

    pltpu.emit_pipeline(
        body1a,
        grid=(_K,),
        out_specs=[pl.BlockSpec((1, _BC, _H, _W), _blk)],
        _explicit_indices=True,
    )(q_hbm)

    def body1b(x_vmem, o_vmem):
        o_vmem[...] = quant(x_vmem[...])

    _H2 = _BC // 2
    _GJ2 = _C // _H2

    def _blk2(n):
        m = n + 2 * _K
        return (m // _GJ2, m % _GJ2, 0, 0)

    pltpu.emit_pipeline(
        body1b,
        grid=(2 * (_N - _K),),
        in_specs=[pl.BlockSpec((1, _H2, _H, _W), _blk2,
                               pipeline_mode=pl.Buffered(buffer_count=4))],
        out_specs=[pl.BlockSpec((1, _H2, _H, _W), _blk2)],
    )(x_hbm, q_hbm)


def kernel(x_ll):
    q, x_min, x_max = pl.pallas_call(
        _outer,
        in_specs=[pl.BlockSpec(memory_space=pl.ANY)],
        out_specs=[
            pl.BlockSpec(memory_space=pl.ANY),
            pl.BlockSpec(memory_space=pltpu.SMEM),
            pl.BlockSpec(memory_space=pltpu.SMEM),
        ],
        out_shape=[
            jax.ShapeDtypeStruct((_B, _C, _H, _W), jnp.float32),
            jax.ShapeDtypeStruct((1, 1), jnp.float32),
            jax.ShapeDtypeStruct((1, 1), jnp.float32),
        ],
        scratch_shapes=[
            pltpu.VMEM((_K, _BC, _H, _W), jnp.float32),
            pltpu.VMEM((1, _W), jnp.float32),
            pltpu.VMEM((1, _W), jnp.float32),
        ],
    )(x_ll)

    return (q, x_min.reshape(()), x_max.reshape(()))
